# Initial kernel scaffold; baseline (speedup 1.0000x reference)
import jax
import jax.numpy as jnp
from jax.experimental import pallas as pl


def kernel(X, edge_index, edge_weight, H, C, W_i, b_i, Wc_i, bc_i, W_f, b_f, Wc_f, bc_f, W_c, b_c, Wc_c, bc_c, W_o, b_o, Wc_o, bc_o):
    raise NotImplementedError("write your pallas kernel here")


# speedup vs baseline: 6.4569x; 6.4569x over previous
import functools

import jax
import jax.numpy as jnp
from jax import lax
from jax.experimental import pallas as pl
from jax.experimental.pallas import tpu as pltpu
from jax.experimental.pallas import tpu_sc as plsc

N = 10000
D = 128
E = 320000
NC = 2
NS = 16
NW = NC * NS
B = 128
NCH = 80
EPT = NCH * B
EPAD = NW * EPT
NCH2 = 2 * NCH
RPT = 632
NP = NS * RPT


def _rsqrt16(x):
    xi = lax.bitcast_convert_type(x, jnp.int32)
    yi = jnp.int32(0x5F3759DF) - lax.shift_right_arithmetic(xi, 1)
    y = lax.bitcast_convert_type(yi, jnp.float32)
    for _ in range(3):
        y = y * (1.5 - 0.5 * x * y * y)
    return jnp.where(x > 0.0, y, 0.0)


def _ab_body(src_hbm, dst_hbm, w_hbm, zn_hbm, ew_hbm,
             src_v, dst_v, w_v, dis_v, ew_v, deg_sh):
    core = lax.axis_index("c")
    sid = lax.axis_index("s")

    @pl.when(sid == 0)
    def _():
        pltpu.sync_copy(zn_hbm, deg_sh)

    pltpu.sync_copy(src_hbm.at[sid], src_v)
    pltpu.sync_copy(dst_hbm.at[sid], dst_v)
    pltpu.sync_copy(w_hbm.at[sid], w_v)
    plsc.subcore_barrier()

    def mask_chunk(c, _):
        for g in range(B // 16):
            sl = pl.ds(g * 16, 16)
            s16 = src_v[c, sl]
            d16 = dst_v[c, sl]
            w_v[c, sl] = jnp.where(s16 != d16, w_v[c, sl], 0.0)
        return 0

    lax.fori_loop(0, NCH2, mask_chunk, 0, unroll=False)

    def deg_chunk(c, _):
        pltpu.sync_copy(w_v.at[c], deg_sh.at[src_v.at[c]], add=True)
        return 0

    lax.fori_loop(0, NCH2, deg_chunk, 0, unroll=False)
    plsc.subcore_barrier()

    pltpu.sync_copy(deg_sh, dis_v)

    def dis_group(i, _):
        sl = pl.ds(i * 16, 16)
        dis_v[sl] = _rsqrt16(dis_v[sl])
        return 0

    lax.fori_loop(0, N // 16, dis_group, 0, unroll=False)

    def ew_chunk(cc, _):
        c = core * NCH + cc
        for g in range(B // 16):
            sl = pl.ds(g * 16, 16)
            s16 = src_v[c, sl]
            d16 = dst_v[c, sl]
            w16 = w_v[c, sl]
            a = plsc.load_gather(dis_v, [s16])
            b = plsc.load_gather(dis_v, [d16])
            ew_v[cc, sl] = -(a * w16 * b)
        return 0

    lax.fori_loop(0, NCH, ew_chunk, 0, unroll=False)
    pltpu.sync_copy(ew_v, ew_hbm.at[sid * NC + core])


def _prop_body(table_hbm, src_hbm, dst_hbm, ew_hbm, znd_hbm, part_hbm,
               sidx_v, didx_v, ew_v, rows_v, acc_sh, sem):
    core = lax.axis_index("c")
    sid = lax.axis_index("s")
    jslot = sid * NC + core

    pltpu.sync_copy(znd_hbm.at[pl.ds(sid * RPT, RPT)],
                    acc_sh.at[pl.ds(sid * RPT, RPT)])
    pltpu.sync_copy(src_hbm.at[jslot], sidx_v)
    pltpu.sync_copy(dst_hbm.at[jslot], didx_v)
    pltpu.sync_copy(ew_hbm.at[jslot], ew_v)
    plsc.subcore_barrier()

    def chunk(c, _):
        pltpu.async_copy(table_hbm.at[sidx_v.at[c]], rows_v, sem).wait()

        def scale_group(g, _):
            ew16 = ew_v[c, pl.ds(g * 16, 16)]
            for e in range(16):
                r = g * 16 + e
                wb = jnp.broadcast_to(ew16[e], (16,))
                for j in range(D // 16):
                    sl = pl.ds(j * 16, 16)
                    rows_v[r, sl] = rows_v[r, sl] * wb
            return 0

        lax.fori_loop(0, B // 16, scale_group, 0, unroll=False)
        pltpu.sync_copy(rows_v, acc_sh.at[didx_v.at[c]], add=True)
        return 0

    lax.fori_loop(0, NCH, chunk, 0, unroll=False)
    plsc.subcore_barrier()
    pltpu.sync_copy(acc_sh.at[pl.ds(sid * RPT, RPT)],
                    part_hbm.at[pl.ds(core * NP + sid * RPT, RPT)])


def _make_sc_kernels():
    mesh = plsc.VectorSubcoreMesh(core_axis_name="c", subcore_axis_name="s",
                                  num_cores=NC, num_subcores=NS)
    ab = functools.partial(
        pl.kernel, _ab_body,
        out_type=jax.ShapeDtypeStruct((NW, NCH, B), jnp.float32),
        mesh=mesh,
        compiler_params=pltpu.CompilerParams(needs_layout_passes=False),
        scratch_types=[
            pltpu.VMEM((NCH2, B), jnp.int32),
            pltpu.VMEM((NCH2, B), jnp.int32),
            pltpu.VMEM((NCH2, B), jnp.float32),
            pltpu.VMEM((N,), jnp.float32),
            pltpu.VMEM((NCH, B), jnp.float32),
            pltpu.VMEM_SHARED((N,), jnp.float32),
        ],
        name="gclstm_deg_ew",
    )()
    prop = functools.partial(
        pl.kernel, _prop_body,
        out_type=jax.ShapeDtypeStruct((NC * NP, D), jnp.float32),
        mesh=mesh,
        compiler_params=pltpu.CompilerParams(needs_layout_passes=False),
        scratch_types=[
            pltpu.VMEM((NCH, B), jnp.int32),
            pltpu.VMEM((NCH, B), jnp.int32),
            pltpu.VMEM((NCH, B), jnp.float32),
            pltpu.VMEM((B, D), jnp.float32),
            pltpu.VMEM_SHARED((NP, D), jnp.float32),
            pltpu.SemaphoreType.DMA,
        ],
        name="gclstm_prop",
    )()
    return ab, prop


_AB_KERNEL, _PROP_KERNEL = _make_sc_kernels()


def _combine_body(a_ref, b_ref, o_ref):
    o_ref[...] = a_ref[...] + b_ref[...]


def _gates_body(x_ref, h_ref, t1_ref, p2a_ref, p2b_ref, c_ref,
                wx_ref, a0_ref, a1_ref, a2_ref, bias_ref,
                hn_ref, cn_ref):
    p2 = 2.0 * (p2a_ref[...] + p2b_ref[...])
    z = jnp.dot(x_ref[...], wx_ref[...], preferred_element_type=jnp.float32)
    z = z + jnp.dot(h_ref[...], a0_ref[...], preferred_element_type=jnp.float32)
    z = z + jnp.dot(t1_ref[...], a1_ref[...], preferred_element_type=jnp.float32)
    z = z + jnp.dot(p2, a2_ref[...], preferred_element_type=jnp.float32)
    z = z + bias_ref[0:1, :]
    gi = jax.nn.sigmoid(z[:, 0 * D:1 * D])
    gf = jax.nn.sigmoid(z[:, 1 * D:2 * D])
    gt = jnp.tanh(z[:, 2 * D:3 * D])
    go = jax.nn.sigmoid(z[:, 3 * D:4 * D])
    cn = gf * c_ref[...] + gi * gt
    hn_ref[...] = go * jnp.tanh(cn)
    cn_ref[...] = cn


_RB = 400
_GRID = N // _RB


def _combine(a, b):
    return pl.pallas_call(
        _combine_body,
        grid=(NS,),
        in_specs=[pl.BlockSpec((RPT, D), lambda i: (i, 0)),
                  pl.BlockSpec((RPT, D), lambda i: (i, 0))],
        out_specs=pl.BlockSpec((RPT, D), lambda i: (i, 0)),
        out_shape=jax.ShapeDtypeStruct((NP, D), jnp.float32),
    )(a, b)


def _gates(x, h, t1, p2a, p2b, c, wx, a0, a1, a2, bias):
    row = pl.BlockSpec((_RB, D), lambda i: (i, 0))
    wsp = pl.BlockSpec((D, 4 * D), lambda i: (0, 0))
    bsp = pl.BlockSpec((8, 4 * D), lambda i: (0, 0))
    return pl.pallas_call(
        _gates_body,
        grid=(_GRID,),
        in_specs=[row, row, row, row, row, row, wsp, wsp, wsp, wsp, bsp],
        out_specs=[row, row],
        out_shape=[jax.ShapeDtypeStruct((N, D), jnp.float32),
                   jax.ShapeDtypeStruct((N, D), jnp.float32)],
    )(x, h, t1, p2a, p2b, c, wx, a0, a1, a2, bias)


def kernel(X, edge_index, edge_weight, H, C,
           W_i, b_i, Wc_i, bc_i,
           W_f, b_f, Wc_f, bc_f,
           W_c, b_c, Wc_c, bc_c,
           W_o, b_o, Wc_o, bc_o):
    src = edge_index[0].astype(jnp.int32)
    dst = edge_index[1].astype(jnp.int32)
    w = edge_weight.astype(jnp.float32)
    pad = EPAD - E
    src = jnp.concatenate([src, jnp.zeros((pad,), jnp.int32)])
    dst = jnp.concatenate([dst, jnp.zeros((pad,), jnp.int32)])
    w = jnp.concatenate([w, jnp.zeros((pad,), jnp.float32)])

    src_ab = src.reshape(NS, NCH2, B)
    dst_ab = dst.reshape(NS, NCH2, B)
    w_ab = w.reshape(NS, NCH2, B)
    src_c = src.reshape(NW, NCH, B)
    dst_c = dst.reshape(NW, NCH, B)

    zn = jnp.zeros((N,), jnp.float32)
    znd = jnp.zeros((NP, D), jnp.float32)

    ew3 = _AB_KERNEL(src_ab, dst_ab, w_ab, zn)

    part1 = _PROP_KERNEL(H, src_c, dst_c, ew3, znd)
    t1 = _combine(part1[:NP], part1[NP:])
    part2 = _PROP_KERNEL(t1, src_c, dst_c, ew3, znd)

    wx = jnp.concatenate([W_i, W_f, W_c, W_o], axis=1)
    a0 = jnp.concatenate([Wc_i[0] - Wc_i[2], Wc_f[0] - Wc_f[2],
                          Wc_c[0] - Wc_c[2], Wc_o[0] - Wc_o[2]], axis=1)
    a1 = jnp.concatenate([Wc_i[1], Wc_f[1], Wc_c[1], Wc_o[1]], axis=1)
    a2 = jnp.concatenate([Wc_i[2], Wc_f[2], Wc_c[2], Wc_o[2]], axis=1)
    bias = jnp.concatenate([b_i + bc_i, b_f + bc_f, b_c + bc_c, b_o + bc_o],
                           axis=1)
    bias = jnp.broadcast_to(bias, (8, 4 * D))

    h_new, c_new = _gates(X, H, t1[:N], part2[:N], part2[NP:NP + N], C,
                          wx, a0, a1, a2, bias)
    return (h_new, c_new)


# speedup vs baseline: 7.6265x; 1.1811x over previous
import functools

import jax
import jax.numpy as jnp
from jax import lax
from jax.experimental import pallas as pl
from jax.experimental.pallas import tpu as pltpu
from jax.experimental.pallas import tpu_sc as plsc

N = 10000
D = 128
E = 320000
NC = 2
NS = 16
NW = NC * NS
B = 128
NCH = 80
EPT = NCH * B
EPAD = NW * EPT
NCH2 = 2 * NCH
RPT = 632
NP = NS * RPT
NCHH = NCH // 2


def _rsqrt16(x):
    xi = lax.bitcast_convert_type(x, jnp.int32)
    yi = jnp.int32(0x5F3759DF) - lax.shift_right_arithmetic(xi, 1)
    y = lax.bitcast_convert_type(yi, jnp.float32)
    for _ in range(3):
        y = y * (1.5 - 0.5 * x * y * y)
    return jnp.where(x > 0.0, y, 0.0)


def _ab_body(src_hbm, dst_hbm, w_hbm, zn_hbm, ew_hbm,
             src_v, dst_v, w_v, dis_v, ew_v, deg_sh):
    core = lax.axis_index("c")
    sid = lax.axis_index("s")

    @pl.when(sid == 0)
    def _():
        pltpu.sync_copy(zn_hbm, deg_sh)

    pltpu.sync_copy(src_hbm.at[sid], src_v)
    pltpu.sync_copy(dst_hbm.at[sid], dst_v)
    pltpu.sync_copy(w_hbm.at[sid], w_v)
    plsc.subcore_barrier()

    def mask_chunk(c, _):
        for g in range(B // 16):
            sl = pl.ds(g * 16, 16)
            s16 = src_v[c, sl]
            d16 = dst_v[c, sl]
            w_v[c, sl] = jnp.where(s16 != d16, w_v[c, sl], 0.0)
        return 0

    lax.fori_loop(0, NCH2, mask_chunk, 0, unroll=False)

    def deg_chunk(c, _):
        pltpu.sync_copy(w_v.at[c], deg_sh.at[src_v.at[c]], add=True)
        return 0

    lax.fori_loop(0, NCH2, deg_chunk, 0, unroll=False)
    plsc.subcore_barrier()

    pltpu.sync_copy(deg_sh, dis_v)

    def dis_group(i, _):
        sl = pl.ds(i * 16, 16)
        dis_v[sl] = _rsqrt16(dis_v[sl])
        return 0

    lax.fori_loop(0, N // 16, dis_group, 0, unroll=False)

    def ew_chunk(cc, _):
        c = core * NCH + cc
        for g in range(B // 16):
            sl = pl.ds(g * 16, 16)
            s16 = src_v[c, sl]
            d16 = dst_v[c, sl]
            w16 = w_v[c, sl]
            a = plsc.load_gather(dis_v, [s16])
            b = plsc.load_gather(dis_v, [d16])
            ew_v[cc, sl] = -(a * w16 * b)
        return 0

    lax.fori_loop(0, NCH, ew_chunk, 0, unroll=False)
    pltpu.sync_copy(ew_v, ew_hbm.at[sid * NC + core])


def _prop_body(table_hbm, src_hbm, dst_hbm, ew_hbm, znd_hbm, part_hbm,
               sidx_v, didx_v, ew_v, rows0_v, rows1_v, acc_sh, sem0, sem1):
    core = lax.axis_index("c")
    sid = lax.axis_index("s")
    jslot = sid * NC + core

    pltpu.sync_copy(znd_hbm.at[pl.ds(sid * RPT, RPT)],
                    acc_sh.at[pl.ds(sid * RPT, RPT)])
    plsc.subcore_barrier()

    def scale(rows_v, c):
        def scale_group(g, _):
            ew16 = ew_v[c, pl.ds(g * 16, 16)]
            for e in range(16):
                r = g * 16 + e
                wb = jnp.broadcast_to(ew16[e], (16,))
                for j in range(D // 16):
                    sl = pl.ds(j * 16, 16)
                    rows_v[r, sl] = rows_v[r, sl] * wb
            return 0

        lax.fori_loop(0, B // 16, scale_group, 0, unroll=False)

    for h in range(2):
        pltpu.sync_copy(src_hbm.at[jslot, pl.ds(h * NCHH, NCHH)], sidx_v)
        pltpu.sync_copy(dst_hbm.at[jslot, pl.ds(h * NCHH, NCHH)], didx_v)
        pltpu.sync_copy(ew_hbm.at[jslot, pl.ds(h * NCHH, NCHH)], ew_v)
        pltpu.async_copy(table_hbm.at[sidx_v.at[0]], rows0_v, sem0)

        def pair(cc, _):
            for par, (mine, nxt, msem, nsem) in enumerate(
                    ((rows0_v, rows1_v, sem0, sem1),
                     (rows1_v, rows0_v, sem1, sem0))):
                c = 2 * cc + par

                @pl.when(c + 1 < NCHH)
                def _():
                    pltpu.async_copy(table_hbm.at[sidx_v.at[c + 1]], nxt, nsem)

                pltpu.make_async_copy(znd_hbm.at[pl.ds(0, B)], mine,
                                      msem).wait()
                scale(mine, c)
                pltpu.sync_copy(mine, acc_sh.at[didx_v.at[c]], add=True)
            return 0

        lax.fori_loop(0, NCHH // 2, pair, 0, unroll=False)
    plsc.subcore_barrier()
    pltpu.sync_copy(acc_sh.at[pl.ds(sid * RPT, RPT)],
                    part_hbm.at[pl.ds(core * NP + sid * RPT, RPT)])


def _make_sc_kernels():
    mesh = plsc.VectorSubcoreMesh(core_axis_name="c", subcore_axis_name="s",
                                  num_cores=NC, num_subcores=NS)
    ab = functools.partial(
        pl.kernel, _ab_body,
        out_type=jax.ShapeDtypeStruct((NW, NCH, B), jnp.float32),
        mesh=mesh,
        compiler_params=pltpu.CompilerParams(needs_layout_passes=False),
        scratch_types=[
            pltpu.VMEM((NCH2, B), jnp.int32),
            pltpu.VMEM((NCH2, B), jnp.int32),
            pltpu.VMEM((NCH2, B), jnp.float32),
            pltpu.VMEM((N,), jnp.float32),
            pltpu.VMEM((NCH, B), jnp.float32),
            pltpu.VMEM_SHARED((N,), jnp.float32),
        ],
        name="gclstm_deg_ew",
    )()
    prop = functools.partial(
        pl.kernel, _prop_body,
        out_type=jax.ShapeDtypeStruct((NC * NP, D), jnp.float32),
        mesh=mesh,
        compiler_params=pltpu.CompilerParams(needs_layout_passes=False),
        scratch_types=[
            pltpu.VMEM((NCHH, B), jnp.int32),
            pltpu.VMEM((NCHH, B), jnp.int32),
            pltpu.VMEM((NCHH, B), jnp.float32),
            pltpu.VMEM((B, D), jnp.float32),
            pltpu.VMEM((B, D), jnp.float32),
            pltpu.VMEM_SHARED((NP, D), jnp.float32),
            pltpu.SemaphoreType.DMA,
            pltpu.SemaphoreType.DMA,
        ],
        name="gclstm_prop",
    )()
    return ab, prop


_AB_KERNEL, _PROP_KERNEL = _make_sc_kernels()


def _combine_body(a_ref, b_ref, o_ref):
    o_ref[...] = a_ref[...] + b_ref[...]


def _gates_body(x_ref, h_ref, t1_ref, p2a_ref, p2b_ref, c_ref,
                wx_ref, a0_ref, a1_ref, a2_ref, bias_ref,
                hn_ref, cn_ref):
    p2 = 2.0 * (p2a_ref[...] + p2b_ref[...])
    z = jnp.dot(x_ref[...], wx_ref[...], preferred_element_type=jnp.float32)
    z = z + jnp.dot(h_ref[...], a0_ref[...], preferred_element_type=jnp.float32)
    z = z + jnp.dot(t1_ref[...], a1_ref[...], preferred_element_type=jnp.float32)
    z = z + jnp.dot(p2, a2_ref[...], preferred_element_type=jnp.float32)
    z = z + bias_ref[0:1, :]
    gi = jax.nn.sigmoid(z[:, 0 * D:1 * D])
    gf = jax.nn.sigmoid(z[:, 1 * D:2 * D])
    gt = jnp.tanh(z[:, 2 * D:3 * D])
    go = jax.nn.sigmoid(z[:, 3 * D:4 * D])
    cn = gf * c_ref[...] + gi * gt
    hn_ref[...] = go * jnp.tanh(cn)
    cn_ref[...] = cn


_RB = 400
_GRID = N // _RB


def _combine(a, b):
    return pl.pallas_call(
        _combine_body,
        grid=(NS,),
        in_specs=[pl.BlockSpec((RPT, D), lambda i: (i, 0)),
                  pl.BlockSpec((RPT, D), lambda i: (i, 0))],
        out_specs=pl.BlockSpec((RPT, D), lambda i: (i, 0)),
        out_shape=jax.ShapeDtypeStruct((NP, D), jnp.float32),
    )(a, b)


def _gates(x, h, t1, p2a, p2b, c, wx, a0, a1, a2, bias):
    row = pl.BlockSpec((_RB, D), lambda i: (i, 0))
    wsp = pl.BlockSpec((D, 4 * D), lambda i: (0, 0))
    bsp = pl.BlockSpec((8, 4 * D), lambda i: (0, 0))
    return pl.pallas_call(
        _gates_body,
        grid=(_GRID,),
        in_specs=[row, row, row, row, row, row, wsp, wsp, wsp, wsp, bsp],
        out_specs=[row, row],
        out_shape=[jax.ShapeDtypeStruct((N, D), jnp.float32),
                   jax.ShapeDtypeStruct((N, D), jnp.float32)],
    )(x, h, t1, p2a, p2b, c, wx, a0, a1, a2, bias)


def kernel(X, edge_index, edge_weight, H, C,
           W_i, b_i, Wc_i, bc_i,
           W_f, b_f, Wc_f, bc_f,
           W_c, b_c, Wc_c, bc_c,
           W_o, b_o, Wc_o, bc_o):
    src = edge_index[0].astype(jnp.int32)
    dst = edge_index[1].astype(jnp.int32)
    w = edge_weight.astype(jnp.float32)
    pad = EPAD - E
    src = jnp.concatenate([src, jnp.zeros((pad,), jnp.int32)])
    dst = jnp.concatenate([dst, jnp.zeros((pad,), jnp.int32)])
    w = jnp.concatenate([w, jnp.zeros((pad,), jnp.float32)])

    src_ab = src.reshape(NS, NCH2, B)
    dst_ab = dst.reshape(NS, NCH2, B)
    w_ab = w.reshape(NS, NCH2, B)
    src_c = src.reshape(NW, NCH, B)
    dst_c = dst.reshape(NW, NCH, B)

    zn = jnp.zeros((N,), jnp.float32)
    znd = jnp.zeros((NP, D), jnp.float32)

    ew3 = _AB_KERNEL(src_ab, dst_ab, w_ab, zn)

    part1 = _PROP_KERNEL(H, src_c, dst_c, ew3, znd)
    t1 = _combine(part1[:NP], part1[NP:])
    part2 = _PROP_KERNEL(t1, src_c, dst_c, ew3, znd)

    wx = jnp.concatenate([W_i, W_f, W_c, W_o], axis=1)
    a0 = jnp.concatenate([Wc_i[0] - Wc_i[2], Wc_f[0] - Wc_f[2],
                          Wc_c[0] - Wc_c[2], Wc_o[0] - Wc_o[2]], axis=1)
    a1 = jnp.concatenate([Wc_i[1], Wc_f[1], Wc_c[1], Wc_o[1]], axis=1)
    a2 = jnp.concatenate([Wc_i[2], Wc_f[2], Wc_c[2], Wc_o[2]], axis=1)
    bias = jnp.concatenate([b_i + bc_i, b_f + bc_f, b_c + bc_c, b_o + bc_o],
                           axis=1)
    bias = jnp.broadcast_to(bias, (8, 4 * D))

    h_new, c_new = _gates(X, H, t1[:N], part2[:N], part2[NP:NP + N], C,
                          wx, a0, a1, a2, bias)
    return (h_new, c_new)


# speedup vs baseline: 9.2209x; 1.2091x over previous
import functools

import jax
import jax.numpy as jnp
from jax import lax
from jax.experimental import pallas as pl
from jax.experimental.pallas import tpu as pltpu
from jax.experimental.pallas import tpu_sc as plsc

N = 10000
D = 128
E = 320000
NC = 2
NS = 16
NW = NC * NS
B = 128
NCH = 80
EPT = NCH * B
EPAD = NW * EPT
NCH2 = 2 * NCH
RPT = 632
NP = NS * RPT
NCHH = NCH // 2
KFAST = 120


def _rsqrt16(x):
    xi = lax.bitcast_convert_type(x, jnp.int32)
    yi = jnp.int32(0x5F3759DF) - lax.shift_right_arithmetic(xi, 1)
    y = lax.bitcast_convert_type(yi, jnp.float32)
    for _ in range(3):
        y = y * (1.5 - 0.5 * x * y * y)
    return jnp.where(x > 0.0, y, 0.0)


def _ab_body(src_hbm, dst_hbm, w_hbm, zn_hbm, ew_hbm,
             src_v, dst_v, w_v, dis_v, ew_v, deg_sh):
    core = lax.axis_index("c")
    sid = lax.axis_index("s")

    @pl.when(sid == 0)
    def _():
        pltpu.sync_copy(zn_hbm, deg_sh)

    pltpu.sync_copy(src_hbm.at[sid], src_v)
    pltpu.sync_copy(dst_hbm.at[sid], dst_v)
    pltpu.sync_copy(w_hbm.at[sid], w_v)
    plsc.subcore_barrier()

    def mask_chunk(c, _):
        for g in range(B // 16):
            sl = pl.ds(g * 16, 16)
            s16 = src_v[c, sl]
            d16 = dst_v[c, sl]
            w_v[c, sl] = jnp.where(s16 != d16, w_v[c, sl], 0.0)
        return 0

    lax.fori_loop(0, NCH2, mask_chunk, 0, unroll=False)

    def deg_chunk(c, _):
        pltpu.sync_copy(w_v.at[c], deg_sh.at[src_v.at[c]], add=True)
        return 0

    lax.fori_loop(0, NCH2, deg_chunk, 0, unroll=False)
    plsc.subcore_barrier()

    pltpu.sync_copy(deg_sh, dis_v)

    def dis_group(i, _):
        sl = pl.ds(i * 16, 16)
        dis_v[sl] = _rsqrt16(dis_v[sl])
        return 0

    lax.fori_loop(0, N // 16, dis_group, 0, unroll=False)

    def ew_chunk(cc, _):
        c = core * NCH + cc
        for g in range(B // 16):
            sl = pl.ds(g * 16, 16)
            s16 = src_v[c, sl]
            d16 = dst_v[c, sl]
            w16 = w_v[c, sl]
            a = plsc.load_gather(dis_v, [s16])
            b = plsc.load_gather(dis_v, [d16])
            ew_v[cc, sl] = -(a * w16 * b)
        return 0

    lax.fori_loop(0, NCH, ew_chunk, 0, unroll=False)
    off = pl.multiple_of(core * NCH, 8)
    pltpu.sync_copy(ew_v, ew_hbm.at[sid, pl.ds(off, NCH)])


def _prop_body(table_hbm, src_hbm, dst_hbm, ew_hbm, znd_hbm, part_hbm,
               sidx_v, didx_v, ew_v, rows0_v, rows1_v, acc_sh, sem0, sem1):
    core = lax.axis_index("c")
    sid = lax.axis_index("s")

    pltpu.sync_copy(znd_hbm.at[pl.ds(sid * RPT, RPT)],
                    acc_sh.at[pl.ds(sid * RPT, RPT)])
    plsc.subcore_barrier()

    def scale(rows_v, c):
        def scale_group(g, _):
            ew16 = ew_v[c, pl.ds(g * 16, 16)]
            for e in range(16):
                r = g * 16 + e
                wb = jnp.broadcast_to(ew16[e], (16,))
                for j in range(D // 16):
                    sl = pl.ds(j * 16, 16)
                    rows_v[r, sl] = rows_v[r, sl] * wb
            return 0

        lax.fori_loop(0, B // 16, scale_group, 0, unroll=False)

    base = jnp.where(core == 0, 0, KFAST)
    nwin = jnp.where(core == 0, KFAST // NCHH, (NCH2 - KFAST) // NCHH)

    def window(wi, _):
        woff = pl.multiple_of(base + wi * NCHH, 8)
        pltpu.sync_copy(src_hbm.at[sid, pl.ds(woff, NCHH)], sidx_v)
        pltpu.sync_copy(dst_hbm.at[sid, pl.ds(woff, NCHH)], didx_v)
        pltpu.sync_copy(ew_hbm.at[sid, pl.ds(woff, NCHH)], ew_v)
        pltpu.async_copy(table_hbm.at[sidx_v.at[0]], rows0_v, sem0)

        def pair(cc, _):
            for par, (mine, nxt, msem, nsem) in enumerate(
                    ((rows0_v, rows1_v, sem0, sem1),
                     (rows1_v, rows0_v, sem1, sem0))):
                c = 2 * cc + par

                @pl.when(c + 1 < NCHH)
                def _():
                    pltpu.async_copy(table_hbm.at[sidx_v.at[c + 1]], nxt, nsem)

                pltpu.make_async_copy(znd_hbm.at[pl.ds(0, B)], mine,
                                      msem).wait()
                scale(mine, c)
                pltpu.sync_copy(mine, acc_sh.at[didx_v.at[c]], add=True)
            return 0

        lax.fori_loop(0, NCHH // 2, pair, 0, unroll=False)
        return 0

    lax.fori_loop(0, nwin, window, 0, unroll=False)
    plsc.subcore_barrier()
    pltpu.sync_copy(acc_sh.at[pl.ds(sid * RPT, RPT)],
                    part_hbm.at[pl.ds(core * NP + sid * RPT, RPT)])


def _make_sc_kernels():
    mesh = plsc.VectorSubcoreMesh(core_axis_name="c", subcore_axis_name="s",
                                  num_cores=NC, num_subcores=NS)
    ab = functools.partial(
        pl.kernel, _ab_body,
        out_type=jax.ShapeDtypeStruct((NS, NCH2, B), jnp.float32),
        mesh=mesh,
        compiler_params=pltpu.CompilerParams(needs_layout_passes=False),
        scratch_types=[
            pltpu.VMEM((NCH2, B), jnp.int32),
            pltpu.VMEM((NCH2, B), jnp.int32),
            pltpu.VMEM((NCH2, B), jnp.float32),
            pltpu.VMEM((N,), jnp.float32),
            pltpu.VMEM((NCH, B), jnp.float32),
            pltpu.VMEM_SHARED((N,), jnp.float32),
        ],
        name="gclstm_deg_ew",
    )()
    prop = functools.partial(
        pl.kernel, _prop_body,
        out_type=jax.ShapeDtypeStruct((NC * NP, D), jnp.float32),
        mesh=mesh,
        compiler_params=pltpu.CompilerParams(needs_layout_passes=False),
        scratch_types=[
            pltpu.VMEM((NCHH, B), jnp.int32),
            pltpu.VMEM((NCHH, B), jnp.int32),
            pltpu.VMEM((NCHH, B), jnp.float32),
            pltpu.VMEM((B, D), jnp.float32),
            pltpu.VMEM((B, D), jnp.float32),
            pltpu.VMEM_SHARED((NP, D), jnp.float32),
            pltpu.SemaphoreType.DMA,
            pltpu.SemaphoreType.DMA,
        ],
        name="gclstm_prop",
    )()
    return ab, prop


_AB_KERNEL, _PROP_KERNEL = _make_sc_kernels()


def _combine_body(a_ref, b_ref, o_ref):
    o_ref[...] = a_ref[...] + b_ref[...]


def _gates_body(x_ref, h_ref, t1_ref, p2a_ref, p2b_ref, c_ref,
                wx_ref, a0_ref, a1_ref, a2_ref, bias_ref,
                hn_ref, cn_ref):
    p2 = 2.0 * (p2a_ref[...] + p2b_ref[...])
    z = jnp.dot(x_ref[...], wx_ref[...], preferred_element_type=jnp.float32)
    z = z + jnp.dot(h_ref[...], a0_ref[...], preferred_element_type=jnp.float32)
    z = z + jnp.dot(t1_ref[...], a1_ref[...], preferred_element_type=jnp.float32)
    z = z + jnp.dot(p2, a2_ref[...], preferred_element_type=jnp.float32)
    z = z + bias_ref[0:1, :]
    gi = jax.nn.sigmoid(z[:, 0 * D:1 * D])
    gf = jax.nn.sigmoid(z[:, 1 * D:2 * D])
    gt = jnp.tanh(z[:, 2 * D:3 * D])
    go = jax.nn.sigmoid(z[:, 3 * D:4 * D])
    cn = gf * c_ref[...] + gi * gt
    hn_ref[...] = go * jnp.tanh(cn)
    cn_ref[...] = cn


_RB = 400
_GRID = N // _RB


def _combine(a, b):
    return pl.pallas_call(
        _combine_body,
        grid=(NS,),
        in_specs=[pl.BlockSpec((RPT, D), lambda i: (i, 0)),
                  pl.BlockSpec((RPT, D), lambda i: (i, 0))],
        out_specs=pl.BlockSpec((RPT, D), lambda i: (i, 0)),
        out_shape=jax.ShapeDtypeStruct((NP, D), jnp.float32),
    )(a, b)


def _gates(x, h, t1, p2a, p2b, c, wx, a0, a1, a2, bias):
    row = pl.BlockSpec((_RB, D), lambda i: (i, 0))
    wsp = pl.BlockSpec((D, 4 * D), lambda i: (0, 0))
    bsp = pl.BlockSpec((8, 4 * D), lambda i: (0, 0))
    return pl.pallas_call(
        _gates_body,
        grid=(_GRID,),
        in_specs=[row, row, row, row, row, row, wsp, wsp, wsp, wsp, bsp],
        out_specs=[row, row],
        out_shape=[jax.ShapeDtypeStruct((N, D), jnp.float32),
                   jax.ShapeDtypeStruct((N, D), jnp.float32)],
    )(x, h, t1, p2a, p2b, c, wx, a0, a1, a2, bias)


def kernel(X, edge_index, edge_weight, H, C,
           W_i, b_i, Wc_i, bc_i,
           W_f, b_f, Wc_f, bc_f,
           W_c, b_c, Wc_c, bc_c,
           W_o, b_o, Wc_o, bc_o):
    src = edge_index[0].astype(jnp.int32)
    dst = edge_index[1].astype(jnp.int32)
    w = edge_weight.astype(jnp.float32)
    pad = EPAD - E
    src = jnp.concatenate([src, jnp.zeros((pad,), jnp.int32)])
    dst = jnp.concatenate([dst, jnp.zeros((pad,), jnp.int32)])
    w = jnp.concatenate([w, jnp.zeros((pad,), jnp.float32)])

    src_ab = src.reshape(NS, NCH2, B)
    dst_ab = dst.reshape(NS, NCH2, B)
    w_ab = w.reshape(NS, NCH2, B)

    zn = jnp.zeros((N,), jnp.float32)
    znd = jnp.zeros((NP, D), jnp.float32)

    ew3 = _AB_KERNEL(src_ab, dst_ab, w_ab, zn)

    part1 = _PROP_KERNEL(H, src_ab, dst_ab, ew3, znd)
    t1 = _combine(part1[:NP], part1[NP:])
    part2 = _PROP_KERNEL(t1, src_ab, dst_ab, ew3, znd)

    wx = jnp.concatenate([W_i, W_f, W_c, W_o], axis=1)
    a0 = jnp.concatenate([Wc_i[0] - Wc_i[2], Wc_f[0] - Wc_f[2],
                          Wc_c[0] - Wc_c[2], Wc_o[0] - Wc_o[2]], axis=1)
    a1 = jnp.concatenate([Wc_i[1], Wc_f[1], Wc_c[1], Wc_o[1]], axis=1)
    a2 = jnp.concatenate([Wc_i[2], Wc_f[2], Wc_c[2], Wc_o[2]], axis=1)
    bias = jnp.concatenate([b_i + bc_i, b_f + bc_f, b_c + bc_c, b_o + bc_o],
                           axis=1)
    bias = jnp.broadcast_to(bias, (8, 4 * D))

    h_new, c_new = _gates(X, H, t1[:N], part2[:N], part2[NP:NP + N], C,
                          wx, a0, a1, a2, bias)
    return (h_new, c_new)
